# baseline (speedup 1.0000x reference)
import functools

import jax
import jax.numpy as jnp
from jax import lax
from jax.experimental import pallas as pl
from jax.experimental.pallas import tpu as pltpu
from jax.experimental.pallas import tpu_sc as plsc

_C = 81
_N = 20000
_ROWS_TC = 1000
_TILES = 25
_RPT = _N // _TILES
_GPT = 4 * _RPT
_GROWS = _GPT // 128


def _tc_softmax_argmax(cls_ref, score_ref, label_ref):
    x = cls_ref[...]
    m = jnp.max(x, axis=1, keepdims=True)
    e = jnp.exp(x - m)
    s = jnp.sum(e, axis=1, keepdims=True)
    score_ref[...] = e / s
    cols = lax.broadcasted_iota(jnp.int32, x.shape, 1)
    cand = jnp.where(x == m, cols, _C)
    label_ref[...] = jnp.min(cand, axis=1).reshape(1, 1, _ROWS_TC)


def _sc_gather_decode(reg_flat, label_hbm, x1h, y1h, x2h, y2h,
                      o1h, o2h, o3h, o4h,
                      labels_v, idx_v, gath_v,
                      px1, py1, px2, py2,
                      q1, q2, q3, q4, sem):
    wid = lax.axis_index("s") * 2 + lax.axis_index("c")

    @pl.when(wid < _TILES)
    def _():
        base = wid * _RPT
        pltpu.sync_copy(label_hbm.at[pl.ds(base, _RPT)], labels_v)
        pltpu.sync_copy(x1h.at[pl.ds(base, _RPT)], px1)
        pltpu.sync_copy(y1h.at[pl.ds(base, _RPT)], py1)
        pltpu.sync_copy(x2h.at[pl.ds(base, _RPT)], px2)
        pltpu.sync_copy(y2h.at[pl.ds(base, _RPT)], py2)

        lanes = lax.iota(jnp.int32, 16)
        for g in range(_GPT // 16):
            p = g * 16
            j = p // _RPT
            i = p - j * _RPT
            lab = labels_v[pl.ds(i, 16)]
            flat = (base + i + lanes) * (4 * _C) + (j * _C) + lab
            idx_v[g // 8, pl.ds((g % 8) * 16, 16)] = flat

        handles = []
        for k in range(_GROWS):
            handles.append(
                pltpu.async_copy(reg_flat.at[idx_v.at[k]], gath_v.at[k], sem))
        for h in handles:
            h.wait()

        for t in range(_RPT // 16):
            def dval(j):
                p = j * _RPT + t * 16
                return gath_v[p // 128, pl.ds(p % 128, 16)]
            dx = dval(0) * 0.1
            dy = dval(1) * 0.1
            dw = dval(2) * 0.2
            dh = dval(3) * 0.2
            x1 = px1[pl.ds(t * 16, 16)]
            y1 = py1[pl.ds(t * 16, 16)]
            x2 = px2[pl.ds(t * 16, 16)]
            y2 = py2[pl.ds(t * 16, 16)]
            cx = (x1 + x2) * 0.5
            cy = (y1 + y2) * 0.5
            pw = x2 - x1 + 1.0
            ph = y2 - y1 + 1.0
            gx = cx + pw * dx
            gy = cy + ph * dy
            gw = pw * jnp.exp(dw)
            gh = ph * jnp.exp(dh)
            hw = (gw - 1.0) * 0.5
            hh = (gh - 1.0) * 0.5
            q1[pl.ds(t * 16, 16)] = gx - hw
            q2[pl.ds(t * 16, 16)] = gy - hh
            q3[pl.ds(t * 16, 16)] = gx + hw
            q4[pl.ds(t * 16, 16)] = gy + hh

        pltpu.sync_copy(q1, o1h.at[pl.ds(base, _RPT)])
        pltpu.sync_copy(q2, o2h.at[pl.ds(base, _RPT)])
        pltpu.sync_copy(q3, o3h.at[pl.ds(base, _RPT)])
        pltpu.sync_copy(q4, o4h.at[pl.ds(base, _RPT)])


@functools.partial(
    pl.kernel,
    mesh=plsc.VectorSubcoreMesh(core_axis_name="c", subcore_axis_name="s"),
    out_type=[jax.ShapeDtypeStruct((_N,), jnp.float32)] * 4,
    scratch_types=[
        pltpu.VMEM((_RPT,), jnp.int32),
        pltpu.VMEM((_GROWS, 128), jnp.int32),
        pltpu.VMEM((_GROWS, 128), jnp.float32),
        pltpu.VMEM((_RPT,), jnp.float32),
        pltpu.VMEM((_RPT,), jnp.float32),
        pltpu.VMEM((_RPT,), jnp.float32),
        pltpu.VMEM((_RPT,), jnp.float32),
        pltpu.VMEM((_RPT,), jnp.float32),
        pltpu.VMEM((_RPT,), jnp.float32),
        pltpu.VMEM((_RPT,), jnp.float32),
        pltpu.VMEM((_RPT,), jnp.float32),
        pltpu.SemaphoreType.DMA,
    ],
)
def _sc_kernel(*refs):
    _sc_gather_decode(*refs)


def kernel(props, cls_out, reg_out):
    score, label = pl.pallas_call(
        _tc_softmax_argmax,
        grid=(_N // _ROWS_TC,),
        in_specs=[pl.BlockSpec((_ROWS_TC, _C), lambda i: (i, 0))],
        out_specs=[
            pl.BlockSpec((_ROWS_TC, _C), lambda i: (i, 0)),
            pl.BlockSpec((1, 1, _ROWS_TC), lambda i: (i, 0, 0)),
        ],
        out_shape=[
            jax.ShapeDtypeStruct((_N, _C), jnp.float32),
            jax.ShapeDtypeStruct((_N // _ROWS_TC, 1, _ROWS_TC), jnp.int32),
        ],
    )(cls_out)
    label = label.reshape(_N)

    reg_flat = reg_out.reshape(-1)
    o1, o2, o3, o4 = _sc_kernel(
        reg_flat, label, props[0], props[1], props[2], props[3])
    preds = jnp.stack([o1, o2, o3, o4], axis=0)
    return (preds, score, label)


# speedup vs baseline: 1.7166x; 1.7166x over previous
import functools

import jax
import jax.numpy as jnp
from jax import lax
from jax.experimental import pallas as pl
from jax.experimental.pallas import tpu as pltpu
from jax.experimental.pallas import tpu_sc as plsc

_C = 81
_N = 20000
_ROWS_TC = 1000
_TILES = 25
_RPT = _N // _TILES
_CHUNK = 80
_NCHUNK = _RPT // _CHUNK


def _tc_softmax_argmax(cls_ref, score_ref, label_ref):
    x = cls_ref[...]
    m = jnp.max(x, axis=1, keepdims=True)
    e = jnp.exp(x - m)
    s = jnp.sum(e, axis=1, keepdims=True)
    score_ref[...] = e / s
    cols = lax.broadcasted_iota(jnp.int32, x.shape, 1)
    cand = jnp.where(x == m, cols, _C)
    label_ref[...] = jnp.min(cand, axis=1).reshape(1, 1, _ROWS_TC)


def _sc_gather_decode(reg_hbm, label_hbm, x1h, y1h, x2h, y2h,
                      o1h, o2h, o3h, o4h,
                      labels_v, buf0, buf1, gath_v,
                      px1, py1, px2, py2,
                      q1, q2, q3, q4, sem0, sem1):
    wid = lax.axis_index("s") * 2 + lax.axis_index("c")

    @pl.when(wid < _TILES)
    def _():
        base = wid * _RPT
        pltpu.sync_copy(label_hbm.at[pl.ds(base, _RPT)], labels_v)
        pltpu.sync_copy(x1h.at[pl.ds(base, _RPT)], px1)
        pltpu.sync_copy(y1h.at[pl.ds(base, _RPT)], py1)
        pltpu.sync_copy(x2h.at[pl.ds(base, _RPT)], px2)
        pltpu.sync_copy(y2h.at[pl.ds(base, _RPT)], py2)

        bufs = (buf0, buf1)
        sems = (sem0, sem1)
        lanes = lax.iota(jnp.int32, 16)

        def fetch(c):
            pltpu.async_copy(
                reg_hbm.at[pl.ds(base + c * _CHUNK, _CHUNK)],
                bufs[c % 2], sems[c % 2])

        fetch(0)
        for c in range(_NCHUNK):
            if c + 1 < _NCHUNK:
                fetch(c + 1)
            pltpu.make_async_copy(
                reg_hbm.at[pl.ds(base + c * _CHUNK, _CHUNK)],
                bufs[c % 2], sems[c % 2]).wait()
            buf = bufs[c % 2]
            for r in range(_CHUNK // 16):
                rows = r * 16 + lanes
                lab = labels_v[pl.ds(c * _CHUNK + r * 16, 16)]
                for j in range(4):
                    vals = plsc.load_gather(buf, [rows, lab + j * _C])
                    gath_v[pl.ds(j * _RPT + c * _CHUNK + r * 16, 16)] = vals

        for t in range(_RPT // 16):
            def dval(j):
                return gath_v[pl.ds(j * _RPT + t * 16, 16)]
            dx = dval(0) * 0.1
            dy = dval(1) * 0.1
            dw = dval(2) * 0.2
            dh = dval(3) * 0.2
            x1 = px1[pl.ds(t * 16, 16)]
            y1 = py1[pl.ds(t * 16, 16)]
            x2 = px2[pl.ds(t * 16, 16)]
            y2 = py2[pl.ds(t * 16, 16)]
            cx = (x1 + x2) * 0.5
            cy = (y1 + y2) * 0.5
            pw = x2 - x1 + 1.0
            ph = y2 - y1 + 1.0
            gx = cx + pw * dx
            gy = cy + ph * dy
            gw = pw * jnp.exp(dw)
            gh = ph * jnp.exp(dh)
            hw = (gw - 1.0) * 0.5
            hh = (gh - 1.0) * 0.5
            q1[pl.ds(t * 16, 16)] = gx - hw
            q2[pl.ds(t * 16, 16)] = gy - hh
            q3[pl.ds(t * 16, 16)] = gx + hw
            q4[pl.ds(t * 16, 16)] = gy + hh

        pltpu.sync_copy(q1, o1h.at[pl.ds(base, _RPT)])
        pltpu.sync_copy(q2, o2h.at[pl.ds(base, _RPT)])
        pltpu.sync_copy(q3, o3h.at[pl.ds(base, _RPT)])
        pltpu.sync_copy(q4, o4h.at[pl.ds(base, _RPT)])


@functools.partial(
    pl.kernel,
    mesh=plsc.VectorSubcoreMesh(core_axis_name="c", subcore_axis_name="s"),
    compiler_params=pltpu.CompilerParams(needs_layout_passes=False),
    out_type=[jax.ShapeDtypeStruct((_N,), jnp.float32)] * 4,
    scratch_types=[
        pltpu.VMEM((_RPT,), jnp.int32),
        pltpu.VMEM((_CHUNK, 4 * _C), jnp.float32),
        pltpu.VMEM((_CHUNK, 4 * _C), jnp.float32),
        pltpu.VMEM((4 * _RPT,), jnp.float32),
        pltpu.VMEM((_RPT,), jnp.float32),
        pltpu.VMEM((_RPT,), jnp.float32),
        pltpu.VMEM((_RPT,), jnp.float32),
        pltpu.VMEM((_RPT,), jnp.float32),
        pltpu.VMEM((_RPT,), jnp.float32),
        pltpu.VMEM((_RPT,), jnp.float32),
        pltpu.VMEM((_RPT,), jnp.float32),
        pltpu.VMEM((_RPT,), jnp.float32),
        pltpu.SemaphoreType.DMA,
        pltpu.SemaphoreType.DMA,
    ],
)
def _sc_kernel(*refs):
    _sc_gather_decode(*refs)


def kernel(props, cls_out, reg_out):
    score, label = pl.pallas_call(
        _tc_softmax_argmax,
        grid=(_N // _ROWS_TC,),
        in_specs=[pl.BlockSpec((_ROWS_TC, _C), lambda i: (i, 0))],
        out_specs=[
            pl.BlockSpec((_ROWS_TC, _C), lambda i: (i, 0)),
            pl.BlockSpec((1, 1, _ROWS_TC), lambda i: (i, 0, 0)),
        ],
        out_shape=[
            jax.ShapeDtypeStruct((_N, _C), jnp.float32),
            jax.ShapeDtypeStruct((_N // _ROWS_TC, 1, _ROWS_TC), jnp.int32),
        ],
    )(cls_out)
    label = label.reshape(_N)

    o1, o2, o3, o4 = _sc_kernel(
        reg_out, label, props[0], props[1], props[2], props[3])
    preds = jnp.stack([o1, o2, o3, o4], axis=0)
    return (preds, score, label)


# speedup vs baseline: 1.7497x; 1.0193x over previous
import functools

import jax
import jax.numpy as jnp
from jax import lax
from jax.experimental import pallas as pl
from jax.experimental.pallas import tpu as pltpu
from jax.experimental.pallas import tpu_sc as plsc

_C = 81
_N = 20000
_TILES = 25
_RPT = _N // _TILES
_CHUNK = 80
_NCHUNK = _RPT // _CHUNK


def _tc_softmax_argmax(cls_ref, score_ref, label_ref):
    x = cls_ref[...]
    m = jnp.max(x, axis=1, keepdims=True)
    e = jnp.exp(x - m)
    s = jnp.sum(e, axis=1, keepdims=True)
    score_ref[...] = e / s
    cols = lax.broadcasted_iota(jnp.int32, x.shape, 1)
    cand = jnp.where(x == m, cols, _C)
    label_ref[...] = jnp.min(cand, axis=1).reshape(1, 1, _RPT)


def _sc_gather_decode(reg_hbm, label_hbm, x1h, y1h, x2h, y2h,
                      preds_hbm, label_out,
                      labels_v, buf0, buf1, gath_v,
                      px1, py1, px2, py2,
                      q1, q2, q3, q4, sem0, sem1):
    wid = lax.axis_index("s") * 2 + lax.axis_index("c")

    @pl.when(wid < _TILES)
    def _():
        base = wid * _RPT
        pltpu.sync_copy(label_hbm.at[wid, 0], labels_v)
        pltpu.sync_copy(labels_v, label_out.at[pl.ds(base, _RPT)])
        pltpu.sync_copy(x1h.at[pl.ds(base, _RPT)], px1)
        pltpu.sync_copy(y1h.at[pl.ds(base, _RPT)], py1)
        pltpu.sync_copy(x2h.at[pl.ds(base, _RPT)], px2)
        pltpu.sync_copy(y2h.at[pl.ds(base, _RPT)], py2)

        bufs = (buf0, buf1)
        sems = (sem0, sem1)
        lanes = lax.iota(jnp.int32, 16)

        def fetch(c):
            pltpu.async_copy(
                reg_hbm.at[pl.ds(base + c * _CHUNK, _CHUNK)],
                bufs[c % 2], sems[c % 2])

        fetch(0)
        for c in range(_NCHUNK):
            if c + 1 < _NCHUNK:
                fetch(c + 1)
            pltpu.make_async_copy(
                reg_hbm.at[pl.ds(base + c * _CHUNK, _CHUNK)],
                bufs[c % 2], sems[c % 2]).wait()
            buf = bufs[c % 2]
            for r in range(_CHUNK // 16):
                rows = r * 16 + lanes
                lab = labels_v[pl.ds(c * _CHUNK + r * 16, 16)]
                for j in range(4):
                    vals = plsc.load_gather(buf, [rows, lab + j * _C])
                    gath_v[pl.ds(j * _RPT + c * _CHUNK + r * 16, 16)] = vals

        for t in range(_RPT // 16):
            def dval(j):
                return gath_v[pl.ds(j * _RPT + t * 16, 16)]
            dx = dval(0) * 0.1
            dy = dval(1) * 0.1
            dw = dval(2) * 0.2
            dh = dval(3) * 0.2
            x1 = px1[pl.ds(t * 16, 16)]
            y1 = py1[pl.ds(t * 16, 16)]
            x2 = px2[pl.ds(t * 16, 16)]
            y2 = py2[pl.ds(t * 16, 16)]
            cx = (x1 + x2) * 0.5
            cy = (y1 + y2) * 0.5
            pw = x2 - x1 + 1.0
            ph = y2 - y1 + 1.0
            gx = cx + pw * dx
            gy = cy + ph * dy
            gw = pw * jnp.exp(dw)
            gh = ph * jnp.exp(dh)
            hw = (gw - 1.0) * 0.5
            hh = (gh - 1.0) * 0.5
            q1[pl.ds(t * 16, 16)] = gx - hw
            q2[pl.ds(t * 16, 16)] = gy - hh
            q3[pl.ds(t * 16, 16)] = gx + hw
            q4[pl.ds(t * 16, 16)] = gy + hh

        pltpu.sync_copy(q1, preds_hbm.at[pl.ds(0 * _N + base, _RPT)])
        pltpu.sync_copy(q2, preds_hbm.at[pl.ds(1 * _N + base, _RPT)])
        pltpu.sync_copy(q3, preds_hbm.at[pl.ds(2 * _N + base, _RPT)])
        pltpu.sync_copy(q4, preds_hbm.at[pl.ds(3 * _N + base, _RPT)])


@functools.partial(
    pl.kernel,
    mesh=plsc.VectorSubcoreMesh(core_axis_name="c", subcore_axis_name="s"),
    compiler_params=pltpu.CompilerParams(needs_layout_passes=False),
    out_type=[
        jax.ShapeDtypeStruct((4 * _N,), jnp.float32),
        jax.ShapeDtypeStruct((_N,), jnp.int32),
    ],
    scratch_types=[
        pltpu.VMEM((_RPT,), jnp.int32),
        pltpu.VMEM((_CHUNK, 4 * _C), jnp.float32),
        pltpu.VMEM((_CHUNK, 4 * _C), jnp.float32),
        pltpu.VMEM((4 * _RPT,), jnp.float32),
        pltpu.VMEM((_RPT,), jnp.float32),
        pltpu.VMEM((_RPT,), jnp.float32),
        pltpu.VMEM((_RPT,), jnp.float32),
        pltpu.VMEM((_RPT,), jnp.float32),
        pltpu.VMEM((_RPT,), jnp.float32),
        pltpu.VMEM((_RPT,), jnp.float32),
        pltpu.VMEM((_RPT,), jnp.float32),
        pltpu.VMEM((_RPT,), jnp.float32),
        pltpu.SemaphoreType.DMA,
        pltpu.SemaphoreType.DMA,
    ],
)
def _sc_kernel(*refs):
    _sc_gather_decode(*refs)


def kernel(props, cls_out, reg_out):
    score, label = pl.pallas_call(
        _tc_softmax_argmax,
        grid=(_TILES,),
        in_specs=[pl.BlockSpec((_RPT, _C), lambda i: (i, 0))],
        out_specs=[
            pl.BlockSpec((_RPT, _C), lambda i: (i, 0)),
            pl.BlockSpec((1, 1, _RPT), lambda i: (i, 0, 0)),
        ],
        out_shape=[
            jax.ShapeDtypeStruct((_N, _C), jnp.float32),
            jax.ShapeDtypeStruct((_TILES, 1, _RPT), jnp.int32),
        ],
    )(cls_out)

    pflat, label1d = _sc_kernel(
        reg_out, label, props[0], props[1], props[2], props[3])
    return (pflat.reshape(4, _N), score, label1d)


# speedup vs baseline: 4.9400x; 2.8233x over previous
import functools

import jax
import jax.numpy as jnp
from jax import lax
from jax.experimental import pallas as pl
from jax.experimental.pallas import tpu as pltpu
from jax.experimental.pallas import tpu_sc as plsc

_C = 81
_N = 20000
_CB = 128
_MAIN = _N // _CB
_NCH = _MAIN + 1
_TAIL = _N - _MAIN * _CB
_NP = _NCH * _CB
_TCB = 10240
_K1G = -(-_N // _TCB)


def _k1a_argmax(cls_ref, label_ref):
    x = cls_ref[...]
    m = jnp.max(x, axis=0, keepdims=True)
    rows = lax.broadcasted_iota(jnp.int32, x.shape, 0)
    cand = jnp.where(x == m, rows, _C)
    label_ref[...] = jnp.min(cand, axis=0).reshape(1, 1, _TCB)


def _k1b_softmax(cls_ref, score_ref):
    x = cls_ref[...]
    m = jnp.max(x, axis=0, keepdims=True)
    e = jnp.exp(x - m)
    s = jnp.sum(e, axis=0, keepdims=True)
    score_ref[...] = e / s


def _k2_gather(reg_t, label3, deltas, label1d,
               buf0, buf1, tbuf, labv5, dv5, sem0, sem1, seml, semw):
    wid = lax.axis_index("s") * 2 + lax.axis_index("c")
    lanes = lax.iota(jnp.int32, 16)
    bufs = (buf0, buf1)
    sems = (sem0, sem1)

    def lab_src(c):
        return label3.at[c // (_TCB // _CB), 0,
                         pl.ds(_CB * (c % (_TCB // _CB)), _CB)]

    def fetch(k):
        c = wid + 32 * k

        @pl.when(c < _MAIN)
        def _():
            pltpu.async_copy(reg_t.at[:, pl.ds(_CB * c, _CB)],
                             bufs[k % 2], sems[k % 2])

    fetch(0)
    for k in range(5):
        c = wid + 32 * k

        @pl.when(c < _MAIN)
        def _():
            pltpu.async_copy(lab_src(c), labv5.at[k], seml)

    for k in range(5):
        c = wid + 32 * k

        @pl.when(c < _MAIN)
        def _():
            pltpu.make_async_copy(lab_src(c), labv5.at[k], seml).wait()

    for k in range(5):
        c = wid + 32 * k
        if k + 1 < 5:
            fetch(k + 1)

        @pl.when(c < _MAIN)
        def _():
            pltpu.make_async_copy(reg_t.at[:, pl.ds(_CB * c, _CB)],
                                  bufs[k % 2], sems[k % 2]).wait()
            buf = bufs[k % 2]
            for g in range(_CB // 16):
                ln = g * 16 + lanes
                lab = labv5[k, pl.ds(g * 16, 16)]
                for j in range(4):
                    vals = plsc.load_gather(buf, [lab + j * _C, ln])
                    dv5[k, j, pl.ds(g * 16, 16)] = vals
            pltpu.async_copy(dv5.at[k], deltas.at[:, pl.ds(_CB * c, _CB)],
                             semw)
            pltpu.async_copy(labv5.at[k, pl.ds(0, _CB)],
                             label1d.at[pl.ds(_CB * c, _CB)], semw)

    @pl.when(wid == 28)
    def _():
        pltpu.sync_copy(reg_t.at[:, pl.ds(_CB * _MAIN, _TAIL)], tbuf)
        pltpu.sync_copy(lab_src(_MAIN), labv5.at[5, pl.ds(0, _CB)])
        for g in range(_TAIL // 16):
            ln = g * 16 + lanes
            lab = labv5[5, pl.ds(g * 16, 16)]
            for j in range(4):
                vals = plsc.load_gather(tbuf, [lab + j * _C, ln])
                dv5[5, j, pl.ds(g * 16, 16)] = vals
        pltpu.sync_copy(dv5.at[5], deltas.at[:, pl.ds(_CB * _MAIN, _CB)])
        pltpu.sync_copy(labv5.at[5, pl.ds(0, _TAIL)],
                        label1d.at[pl.ds(_CB * _MAIN, _TAIL)])

    for k in range(5):
        c = wid + 32 * k

        @pl.when(c < _MAIN)
        def _():
            pltpu.make_async_copy(dv5.at[k],
                                  deltas.at[:, pl.ds(_CB * c, _CB)],
                                  semw).wait()
            pltpu.make_async_copy(labv5.at[k, pl.ds(0, _CB)],
                                  label1d.at[pl.ds(_CB * c, _CB)],
                                  semw).wait()


@functools.partial(
    pl.kernel,
    mesh=plsc.VectorSubcoreMesh(core_axis_name="c", subcore_axis_name="s"),
    compiler_params=pltpu.CompilerParams(needs_layout_passes=False),
    out_type=[
        jax.ShapeDtypeStruct((4, _NP), jnp.float32),
        jax.ShapeDtypeStruct((_N,), jnp.int32),
    ],
    scratch_types=[
        pltpu.VMEM((4 * _C, _CB), jnp.float32),
        pltpu.VMEM((4 * _C, _CB), jnp.float32),
        pltpu.VMEM((4 * _C, _TAIL), jnp.float32),
        pltpu.VMEM((6, _CB), jnp.int32),
        pltpu.VMEM((6, 4, _CB), jnp.float32),
        pltpu.SemaphoreType.DMA,
        pltpu.SemaphoreType.DMA,
        pltpu.SemaphoreType.DMA,
        pltpu.SemaphoreType.DMA,
    ],
)
def _k2(*refs):
    _k2_gather(*refs)


def _k3_decode(deltas_ref, props_ref, preds_ref):
    d = deltas_ref[:, :_N]
    p = props_ref[...]
    dx = d[0:1] * 0.1
    dy = d[1:2] * 0.1
    dw = d[2:3] * 0.2
    dh = d[3:4] * 0.2
    x1, y1, x2, y2 = p[0:1], p[1:2], p[2:3], p[3:4]
    cx = (x1 + x2) * 0.5
    cy = (y1 + y2) * 0.5
    pw = x2 - x1 + 1.0
    ph = y2 - y1 + 1.0
    gx = cx + pw * dx
    gy = cy + ph * dy
    gw = pw * jnp.exp(dw)
    gh = ph * jnp.exp(dh)
    hw = (gw - 1.0) * 0.5
    hh = (gh - 1.0) * 0.5
    preds_ref[...] = jnp.concatenate(
        [gx - hw, gy - hh, gx + hw, gy + hh], axis=0)


def kernel(props, cls_out, reg_out):
    cls_t = cls_out.T
    label3 = pl.pallas_call(
        _k1a_argmax,
        grid=(_K1G,),
        in_specs=[pl.BlockSpec((_C, _TCB), lambda i: (0, i))],
        out_specs=pl.BlockSpec((1, 1, _TCB), lambda i: (i, 0, 0)),
        out_shape=jax.ShapeDtypeStruct((_K1G, 1, _TCB), jnp.int32),
    )(cls_t)
    score_t = pl.pallas_call(
        _k1b_softmax,
        grid=(_K1G,),
        in_specs=[pl.BlockSpec((_C, _TCB), lambda i: (0, i))],
        out_specs=pl.BlockSpec((_C, _TCB), lambda i: (0, i)),
        out_shape=jax.ShapeDtypeStruct((_C, _N), jnp.float32),
    )(cls_t)

    reg_t = reg_out.T
    deltas, label1d = _k2(reg_t, label3)

    preds = pl.pallas_call(
        _k3_decode,
        in_specs=[
            pl.BlockSpec((4, _NP), lambda: (0, 0)),
            pl.BlockSpec((4, _N), lambda: (0, 0)),
        ],
        out_specs=pl.BlockSpec((4, _N), lambda: (0, 0)),
        out_shape=jax.ShapeDtypeStruct((4, _N), jnp.float32),
    )(deltas, props)

    return (preds, score_t.T, label1d)
